# Initial kernel scaffold; baseline (speedup 1.0000x reference)
import jax
import jax.numpy as jnp
from jax.experimental import pallas as pl


def kernel(p, params):
    raise NotImplementedError("write your pallas kernel here")


# speedup vs baseline: 2.8126x; 2.8126x over previous
import jax
import jax.numpy as jnp
from jax.experimental import pallas as pl

B, N, D = 4, 4096, 3
HID = 129
CD = 129
K = 20
H2 = 2 * HID // 3
H3 = HID // 3
C3 = CD // 3
EPS = 1e-6

ROWS = 256
NEG_BIG = -3.0e38


def _b16(v):
    return v.astype(jnp.bfloat16).astype(jnp.float32)


def _knn_feat_kernel(rows_ref, ptsT_ref, pts_ref, mdiff_ref, mcr_ref):
    rows = rows_ref[0]
    ptsT = ptsT_ref[0]
    pts = pts_ref[0]
    ip = jax.lax.dot_general(rows, ptsT, (((1,), (0,)), ((), ())),
                             preferred_element_type=jnp.float32)
    xx_r = jnp.sum(rows * rows, axis=1, keepdims=True)
    xx_c = jnp.sum(ptsT * ptsT, axis=0, keepdims=True)
    d = 2.0 * ip - xx_r - xx_c

    iota = jax.lax.broadcasted_iota(jnp.int32, (ROWS, N), 1)
    r0 = rows[:, 0:1]
    r1 = rows[:, 1:2]
    r2 = rows[:, 2:3]

    def body(_, carry):
        d, sd, scr = carry
        mx = jnp.max(d, axis=1, keepdims=True)
        sel = jnp.min(jnp.where(d >= mx, iota, N), axis=1, keepdims=True)
        ohb = iota == sel
        oh = ohb.astype(jnp.float32)
        d = jnp.where(ohb, NEG_BIG, d)
        nbr = jax.lax.dot_general(oh, pts, (((1,), (0,)), ((), ())),
                                  preferred_element_type=jnp.float32,
                                  precision=jax.lax.Precision.HIGHEST)
        n0 = nbr[:, 0:1]
        n1 = nbr[:, 1:2]
        n2 = nbr[:, 2:3]
        diff = nbr - rows
        cr = jnp.concatenate([n1 * r2 - n2 * r1,
                              n2 * r0 - n0 * r2,
                              n0 * r1 - n1 * r0], axis=1)
        return d, sd + _b16(diff), scr + _b16(cr)

    zero3 = jnp.zeros((ROWS, 3), jnp.float32)
    _, sd, scr = jax.lax.fori_loop(0, K, body, (d, zero3, zero3))
    mdiff_ref[0] = sd * (1.0 / K)
    mcr_ref[0] = scr * (1.0 / K)


def _knn_feat(p):
    ptsT = p.transpose(0, 2, 1)
    return pl.pallas_call(
        _knn_feat_kernel,
        grid=(B, N // ROWS),
        in_specs=[
            pl.BlockSpec((1, ROWS, 3), lambda b, i: (b, i, 0)),
            pl.BlockSpec((1, 3, N), lambda b, i: (b, 0, 0)),
            pl.BlockSpec((1, N, 3), lambda b, i: (b, 0, 0)),
        ],
        out_specs=[
            pl.BlockSpec((1, ROWS, 3), lambda b, i: (b, i, 0)),
            pl.BlockSpec((1, ROWS, 3), lambda b, i: (b, i, 0)),
        ],
        out_shape=[
            jax.ShapeDtypeStruct((B, N, 3), jnp.float32),
            jax.ShapeDtypeStruct((B, N, 3), jnp.float32),
        ],
    )(p, ptsT, p)


def _mm(W_ref, xs):
    W = W_ref[...]
    return [jax.lax.dot_general(W, x, (((1,), (0,)), ((), ())),
                                preferred_element_type=jnp.float32)
            for x in xs]


def _vn_lrelu(xs, Wd_ref):
    ds = _mm(Wd_ref, xs)
    dot = xs[0] * ds[0] + xs[1] * ds[1] + xs[2] * ds[2]
    dsq = ds[0] * ds[0] + ds[1] * ds[1] + ds[2] * ds[2]
    q = jnp.where(dot >= 0, 0.0, dot / (dsq + EPS))
    return [x - q * d for x, d in zip(xs, ds)]


def _resblock(xs, a0, f0, a1, f1, sc):
    net = _mm(f0, _vn_lrelu(xs, a0))
    dx = _mm(f1, _vn_lrelu(net, a1))
    short = _mm(sc, xs)
    return [s + d for s, d in zip(short, dx)]


def _dense_kernel(ptsT_ref, mdT_ref, mcT_ref, fc_pos,
                  b0a0, b0f0, b0a1, b0f1, b0sc,
                  b1a0, b1f0, b1a1, b1f1, b1sc,
                  b2a0, b2f0, b2a1, b2f1, b2sc,
                  b3a0, b3f0, b3a1, b3f1, b3sc,
                  b4a0, b4f0, b4a1, b4f1, b4sc,
                  actvn, fc_c, out_ref):
    blocks = [(b0a0, b0f0, b0a1, b0f1, b0sc),
              (b1a0, b1f0, b1a1, b1f1, b1sc),
              (b2a0, b2f0, b2a1, b2f1, b2sc),
              (b3a0, b3f0, b3a1, b3f1, b3sc),
              (b4a0, b4f0, b4a1, b4f1, b4sc)]
    pb = _b16(ptsT_ref[0])
    md = mdT_ref[0]
    mc = mcT_ref[0]
    wb = _b16(fc_pos[...])
    xs = [wb[:, 0:1] * md[s:s + 1, :] + wb[:, 1:2] * pb[s:s + 1, :]
          + wb[:, 2:3] * mc[s:s + 1, :] for s in range(3)]
    for i, blk in enumerate(blocks):
        net = _resblock(xs, *blk)
        if i < 4:
            xs = [jnp.concatenate(
                [nt, jnp.broadcast_to(jnp.mean(nt, axis=1, keepdims=True),
                                      nt.shape)], axis=0) for nt in net]
        else:
            xs = net
    vs = [jnp.mean(x, axis=1, keepdims=True) for x in xs]
    vs = _vn_lrelu(vs, actvn)
    cs = _mm(fc_c, vs)
    out_ref[0] = jnp.concatenate(cs, axis=1)


def _dense_net(ptsT, mdT, mcT, params):
    ws = [params['fc_pos']]
    for i in range(5):
        pr = params['block_%d' % i]
        ws += [pr['a0'], pr['f0'], pr['a1'], pr['f1'], pr['sc']]
    ws += [params['actvn'], params['fc_c']]
    w_specs = [pl.BlockSpec(w.shape, lambda b: (0,) * w.ndim) for w in ws]
    return pl.pallas_call(
        _dense_kernel,
        grid=(B,),
        in_specs=[
            pl.BlockSpec((1, 3, N), lambda b: (b, 0, 0)),
            pl.BlockSpec((1, 3, N), lambda b: (b, 0, 0)),
            pl.BlockSpec((1, 3, N), lambda b: (b, 0, 0)),
        ] + w_specs,
        out_specs=pl.BlockSpec((1, C3, 3), lambda b: (b, 0, 0)),
        out_shape=jax.ShapeDtypeStruct((B, C3, 3), jnp.float32),
    )(ptsT, mdT, mcT, *ws)


@jax.jit
def kernel(p, params):
    mdiff, mcr = _knn_feat(p)
    c = _dense_net(p.transpose(0, 2, 1), mdiff.transpose(0, 2, 1),
                   mcr.transpose(0, 2, 1), params)
    return c.reshape(B, -1)


# speedup vs baseline: 4.9015x; 1.7427x over previous
import jax
import jax.numpy as jnp
from jax.experimental import pallas as pl

B, N, D = 4, 4096, 3
HID = 129
CD = 129
K = 20
H2 = 2 * HID // 3
H3 = HID // 3
C3 = CD // 3
EPS = 1e-6

ROWS = 256
NEG_BIG = -3.0e38


def _b16(v):
    return v.astype(jnp.bfloat16).astype(jnp.float32)


def _knn_feat_kernel(rows_ref, rowsT_ref, ptsT_ref, mdiff_ref, mcr_ref):
    rows = rows_ref[0]
    rowsT = rowsT_ref[0]
    ptsT = ptsT_ref[0]
    ip = jax.lax.dot_general(rows, ptsT, (((1,), (0,)), ((), ())),
                             preferred_element_type=jnp.float32)
    xx_r = jnp.sum(rows * rows, axis=1, keepdims=True)
    xx_c = jnp.sum(ptsT * ptsT, axis=0, keepdims=True)
    d = 2.0 * ip - xx_r - xx_c

    iota = jax.lax.broadcasted_iota(jnp.int32, (ROWS, N), 1)

    def body(_, carry):
        d, msk = carry
        mx = jnp.max(d, axis=1, keepdims=True)
        sel = jnp.min(jnp.where(d >= mx, iota, N), axis=1, keepdims=True)
        ohb = iota == sel
        return jnp.where(ohb, NEG_BIG, d), jnp.where(ohb, 1.0, msk)

    _, maskf = jax.lax.fori_loop(
        0, K, body, (d, jnp.zeros((ROWS, N), jnp.float32)))

    p0 = ptsT[0:1, :]
    p1 = ptsT[1:2, :]
    p2 = ptsT[2:3, :]
    x0 = rows[:, 0:1]
    x1 = rows[:, 1:2]
    x2 = rows[:, 2:3]

    def msum(mat):
        return jnp.sum(maskf * _b16(mat), axis=1, keepdims=True) * (1.0 / K)

    mdiff_ref[0] = jnp.concatenate(
        [msum(p0 - x0), msum(p1 - x1), msum(p2 - x2)], axis=1)
    mcr_ref[0] = jnp.concatenate(
        [msum(p1 * x2 - p2 * x1), msum(p2 * x0 - p0 * x2),
         msum(p0 * x1 - p1 * x0)], axis=1)


def _knn_feat(p):
    ptsT = p.transpose(0, 2, 1)
    return pl.pallas_call(
        _knn_feat_kernel,
        grid=(B, N // ROWS),
        in_specs=[
            pl.BlockSpec((1, ROWS, 3), lambda b, i: (b, i, 0)),
            pl.BlockSpec((1, 3, ROWS), lambda b, i: (b, 0, i)),
            pl.BlockSpec((1, 3, N), lambda b, i: (b, 0, 0)),
        ],
        out_specs=[
            pl.BlockSpec((1, ROWS, 3), lambda b, i: (b, i, 0)),
            pl.BlockSpec((1, ROWS, 3), lambda b, i: (b, i, 0)),
        ],
        out_shape=[
            jax.ShapeDtypeStruct((B, N, 3), jnp.float32),
            jax.ShapeDtypeStruct((B, N, 3), jnp.float32),
        ],
    )(p, ptsT, ptsT)


def _mm(W_ref, xs):
    W = W_ref[...]
    return [jax.lax.dot_general(W, x, (((1,), (0,)), ((), ())),
                                preferred_element_type=jnp.float32)
            for x in xs]


def _vn_lrelu(xs, Wd_ref):
    ds = _mm(Wd_ref, xs)
    dot = xs[0] * ds[0] + xs[1] * ds[1] + xs[2] * ds[2]
    dsq = ds[0] * ds[0] + ds[1] * ds[1] + ds[2] * ds[2]
    q = jnp.where(dot >= 0, 0.0, dot / (dsq + EPS))
    return [x - q * d for x, d in zip(xs, ds)]


def _resblock(xs, a0, f0, a1, f1, sc):
    net = _mm(f0, _vn_lrelu(xs, a0))
    dx = _mm(f1, _vn_lrelu(net, a1))
    short = _mm(sc, xs)
    return [s + d for s, d in zip(short, dx)]


def _dense_kernel(ptsT_ref, mdT_ref, mcT_ref, fc_pos,
                  b0a0, b0f0, b0a1, b0f1, b0sc,
                  b1a0, b1f0, b1a1, b1f1, b1sc,
                  b2a0, b2f0, b2a1, b2f1, b2sc,
                  b3a0, b3f0, b3a1, b3f1, b3sc,
                  b4a0, b4f0, b4a1, b4f1, b4sc,
                  actvn, fc_c, out_ref):
    blocks = [(b0a0, b0f0, b0a1, b0f1, b0sc),
              (b1a0, b1f0, b1a1, b1f1, b1sc),
              (b2a0, b2f0, b2a1, b2f1, b2sc),
              (b3a0, b3f0, b3a1, b3f1, b3sc),
              (b4a0, b4f0, b4a1, b4f1, b4sc)]
    pb = _b16(ptsT_ref[0])
    md = mdT_ref[0]
    mc = mcT_ref[0]
    wb = _b16(fc_pos[...])
    xs = [wb[:, 0:1] * md[s:s + 1, :] + wb[:, 1:2] * pb[s:s + 1, :]
          + wb[:, 2:3] * mc[s:s + 1, :] for s in range(3)]
    for i, blk in enumerate(blocks):
        net = _resblock(xs, *blk)
        if i < 4:
            xs = [jnp.concatenate(
                [nt, jnp.broadcast_to(jnp.mean(nt, axis=1, keepdims=True),
                                      nt.shape)], axis=0) for nt in net]
        else:
            xs = net
    vs = [jnp.mean(x, axis=1, keepdims=True) for x in xs]
    vs = _vn_lrelu(vs, actvn)
    cs = _mm(fc_c, vs)
    out_ref[0] = jnp.concatenate(cs, axis=1)


def _dense_net(ptsT, mdT, mcT, params):
    ws = [params['fc_pos']]
    for i in range(5):
        pr = params['block_%d' % i]
        ws += [pr['a0'], pr['f0'], pr['a1'], pr['f1'], pr['sc']]
    ws += [params['actvn'], params['fc_c']]
    w_specs = [pl.BlockSpec(w.shape, lambda b: (0,) * w.ndim) for w in ws]
    return pl.pallas_call(
        _dense_kernel,
        grid=(B,),
        in_specs=[
            pl.BlockSpec((1, 3, N), lambda b: (b, 0, 0)),
            pl.BlockSpec((1, 3, N), lambda b: (b, 0, 0)),
            pl.BlockSpec((1, 3, N), lambda b: (b, 0, 0)),
        ] + w_specs,
        out_specs=pl.BlockSpec((1, C3, 3), lambda b: (b, 0, 0)),
        out_shape=jax.ShapeDtypeStruct((B, C3, 3), jnp.float32),
    )(ptsT, mdT, mcT, *ws)


@jax.jit
def kernel(p, params):
    mdiff, mcr = _knn_feat(p)
    c = _dense_net(p.transpose(0, 2, 1), mdiff.transpose(0, 2, 1),
                   mcr.transpose(0, 2, 1), params)
    return c.reshape(B, -1)
